# baseline (speedup 1.0000x reference)
import functools
import jax
import jax.numpy as jnp
from jax import lax
from jax.experimental import pallas as pl
from jax.experimental.pallas import tpu as pltpu
from jax.experimental.pallas import tpu_sc as plsc

T = 2048
H = 2048
D = 256
E = 8
I = 2048

B = 256
NB = 2 * T // B + E
NPAD = NB * B
IBLK = 512
NC = 2
NS = 16
NW = NC * NS
TOK = T // NW


def _gelu_exact(x):
    return x * 0.5 * (1.0 + lax.erf(x * (2.0 ** -0.5)))


def _router_body(x_ref, wd_ref, bd_ref, rmsw_ref, w1_ref, b1_ref, w2_ref,
                 b2_ref, w3_ref, hs_ref, pe_ref, po_ref, re_ref, ro_ref,
                 bexp_ref, bval_ref):
    x = x_ref[...]
    hs = jnp.dot(x, wd_ref[...], preferred_element_type=jnp.float32)
    hs = hs + bd_ref[...][None, :]
    hs_ref[...] = hs
    ms = jnp.mean(hs * hs, axis=-1, keepdims=True)
    hsn = hs * lax.rsqrt(ms + 1e-5) * rmsw_ref[...][None, :]
    z = _gelu_exact(jnp.dot(hsn, w1_ref[...], preferred_element_type=jnp.float32)
                    + b1_ref[...][None, :])
    z = _gelu_exact(jnp.dot(z, w2_ref[...], preferred_element_type=jnp.float32)
                    + b2_ref[...][None, :])
    logits = jnp.dot(z, w3_ref[...], preferred_element_type=jnp.float32)
    m = jnp.max(logits, axis=-1, keepdims=True)
    ex = jnp.exp(logits - m)
    probs = ex / jnp.sum(ex, axis=-1, keepdims=True)
    eidx = lax.broadcasted_iota(jnp.int32, probs.shape, 1)
    m1 = jnp.max(probs, axis=-1, keepdims=True)
    i1 = jnp.min(jnp.where(probs == m1, eidx, E), axis=-1, keepdims=True)
    sel1 = eidx == i1
    masked = jnp.where(sel1, -jnp.inf, probs)
    m2 = jnp.max(masked, axis=-1, keepdims=True)
    i2 = jnp.min(jnp.where(masked == m2, eidx, E), axis=-1, keepdims=True)
    sel2 = eidx == i2
    re_ref[...] = m1
    ro_ref[...] = m2

    sel = (sel1 | sel2).astype(jnp.float32)
    r_io = lax.broadcasted_iota(jnp.int32, (T, T), 0)
    c_io = lax.broadcasted_iota(jnp.int32, (T, T), 1)
    tri = (c_io < r_io).astype(jnp.float32)
    cum_excl = jnp.dot(tri, sel, preferred_element_type=jnp.float32)
    counts = jnp.sum(sel, axis=0, keepdims=True)
    padded = jnp.ceil(counts / B) * B
    e_r = lax.broadcasted_iota(jnp.int32, (E, E), 0)
    e_c = lax.broadcasted_iota(jnp.int32, (E, E), 1)
    tri8 = (e_r <= e_c).astype(jnp.float32)
    pbase_incl = jnp.dot(padded, tri8, preferred_element_type=jnp.float32)
    pbase = pbase_incl - padded
    pos_mat = pbase + cum_excl
    pe_ref[...] = jnp.sum(jnp.where(sel1, pos_mat, 0.0), axis=-1,
                          keepdims=True).astype(jnp.int32)
    po_ref[...] = jnp.sum(jnp.where(sel2, pos_mat, 0.0), axis=-1,
                          keepdims=True).astype(jnp.int32)

    blk_start = pbase / B
    blk_cnt = padded / B
    e8 = lax.broadcasted_iota(jnp.int32, (1, E), 1).astype(jnp.float32)
    maxe = jnp.max(jnp.where(counts > 0, e8, -1.0))
    used = jnp.sum(blk_cnt)
    b_io = lax.broadcasted_iota(jnp.int32, (1, 128), 1).astype(jnp.float32)
    bexp = jnp.full((1, 128), maxe)
    for e in range(E):
        bs = jnp.sum(jnp.where(e8 == e, blk_start, 0.0))
        bc = jnp.sum(jnp.where(e8 == e, blk_cnt, 0.0))
        bexp = jnp.where((b_io >= bs) & (b_io < bs + bc), float(e), bexp)
    bexp_ref[...] = bexp.astype(jnp.int32)
    bval_ref[...] = (b_io < used).astype(jnp.int32)


def _dispatch_body(hid_ref, pe_ref, po_ref, xs_ref, pev, pov, peo, stage,
                   sem):
    wid = lax.axis_index("s") * NC + lax.axis_index("c")
    t0 = wid * TOK
    pltpu.sync_copy(pe_ref.at[pl.ds(t0, TOK)], pev)
    pltpu.sync_copy(po_ref.at[pl.ds(t0, TOK)], pov)
    nch = TOK // 16
    for c in range(nch):
        peo[c] = pev[pl.ds(16 * c, 16)]
        peo[nch + c] = pov[pl.ds(16 * c, 16)]
    for c in range(nch):
        pltpu.sync_copy(hid_ref.at[pl.ds(t0 + 16 * c, 16)], stage)
        cp1 = pltpu.async_copy(stage, xs_ref.at[peo.at[c]], sem)
        cp2 = pltpu.async_copy(stage, xs_ref.at[peo.at[nch + c]], sem)
        cp1.wait()
        cp2.wait()


def _expert_body(s_ref, x_ref, wg_ref, wu_ref, wo_ref, y_ref):
    b = pl.program_id(0)
    c = pl.program_id(1)

    @pl.when(s_ref[128 + b] == 1)
    def _():
        x = x_ref[...].astype(jnp.bfloat16)
        g = jnp.dot(x, wg_ref[0].astype(jnp.bfloat16),
                    preferred_element_type=jnp.float32)
        u = jnp.dot(x, wu_ref[0].astype(jnp.bfloat16),
                    preferred_element_type=jnp.float32)
        h = (g * jax.nn.sigmoid(g)) * u
        yc = jnp.dot(h.astype(jnp.bfloat16), wo_ref[0].astype(jnp.bfloat16),
                     preferred_element_type=jnp.float32)

        @pl.when(c == 0)
        def _():
            y_ref[...] = yc

        @pl.when(c != 0)
        def _():
            y_ref[...] += yc


_GDIMS = lax.GatherDimensionNumbers(offset_dims=(), collapsed_slice_dims=(0,),
                                    start_index_map=(0,))


def _bcast_lane(vec, lane):
    idx = jnp.full((16, 1), lane, jnp.int32)
    return lax.gather(vec, idx, _GDIMS, (1,),
                      mode=lax.GatherScatterMode.PROMISE_IN_BOUNDS)


def _combine_body(y_ref, pe_ref, po_ref, rp_ref, out_ref, pev, pov, rpv,
                  stg_e, stg_o, acc, sem):
    wid = lax.axis_index("s") * NC + lax.axis_index("c")
    t0 = wid * TOK
    pltpu.sync_copy(pe_ref.at[pl.ds(t0, TOK)], pev)
    pltpu.sync_copy(po_ref.at[pl.ds(t0, TOK)], pov)
    pltpu.sync_copy(rp_ref.at[pl.ds(2 * t0, 2 * TOK)], rpv)
    for c in range(TOK // 8):
        cpe = pltpu.async_copy(y_ref.at[pev.at[pl.ds(8 * c, 8)]], stg_e, sem)
        cpo = pltpu.async_copy(y_ref.at[pov.at[pl.ds(8 * c, 8)]], stg_o, sem)
        cpe.wait()
        cpo.wait()
        rvv = rpv[pl.ds(16 * c, 16)]
        for i in range(8):
            w0 = _bcast_lane(rvv, 2 * i)
            w1 = _bcast_lane(rvv, 2 * i + 1)

            def seg_body(j, _):
                sl = pl.ds(j * 16, 16)
                acc[i, sl] = w0 * stg_e[i, sl] + w1 * stg_o[i, sl]
                return 0

            lax.fori_loop(0, H // 16, seg_body, 0)
        pltpu.sync_copy(acc, out_ref.at[pl.ds(t0 + 8 * c, 8)])


@functools.cache
def _sc_kernels():
    mesh = plsc.VectorSubcoreMesh(core_axis_name="c", subcore_axis_name="s")
    dispatch = pl.kernel(
        _dispatch_body,
        out_type=jax.ShapeDtypeStruct((NPAD, H), jnp.float32),
        mesh=mesh,
        scratch_types=[
            pltpu.VMEM((TOK,), jnp.int32),
            pltpu.VMEM((TOK,), jnp.int32),
            pltpu.VMEM((2 * (TOK // 16), 16), jnp.int32),
            pltpu.VMEM((16, H), jnp.float32),
            pltpu.SemaphoreType.DMA,
        ],
    )
    combine = pl.kernel(
        _combine_body,
        out_type=jax.ShapeDtypeStruct((T, H), jnp.float32),
        mesh=mesh,
        scratch_types=[
            pltpu.VMEM((TOK,), jnp.int32),
            pltpu.VMEM((TOK,), jnp.int32),
            pltpu.VMEM((2 * TOK,), jnp.float32),
            pltpu.VMEM((8, H), jnp.float32),
            pltpu.VMEM((8, H), jnp.float32),
            pltpu.VMEM((8, H), jnp.float32),
            pltpu.SemaphoreType.DMA,
        ],
    )
    return dispatch, combine


@jax.jit
def kernel(hidden_states, W_down, b_down, rms_w, W_r1, b_r1, W_r2, b_r2,
           W_r3, w_gate, w_up, w_out):
    hs, pe, po, re, ro, bexp, bval = pl.pallas_call(
        _router_body,
        out_shape=(
            jax.ShapeDtypeStruct((T, D), jnp.float32),
            jax.ShapeDtypeStruct((T, 1), jnp.int32),
            jax.ShapeDtypeStruct((T, 1), jnp.int32),
            jax.ShapeDtypeStruct((T, 1), jnp.float32),
            jax.ShapeDtypeStruct((T, 1), jnp.float32),
            jax.ShapeDtypeStruct((1, 128), jnp.int32),
            jax.ShapeDtypeStruct((1, 128), jnp.int32),
        ),
    )(hidden_states, W_down, b_down, rms_w, W_r1, b_r1, W_r2, b_r2, W_r3)

    pe1 = pe.reshape(T)
    po1 = po.reshape(T)
    rp = jnp.concatenate([re, ro], axis=1).reshape(2 * T)
    smap = jnp.concatenate([bexp, bval], axis=0).reshape(256)

    dispatch, combine = _sc_kernels()
    xs = dispatch(hidden_states, pe1, po1)

    y = pl.pallas_call(
        _expert_body,
        grid_spec=pltpu.PrefetchScalarGridSpec(
            num_scalar_prefetch=1,
            grid=(NB, I // IBLK),
            in_specs=[
                pl.BlockSpec((B, H), lambda b, c, s: (b, 0)),
                pl.BlockSpec((1, H, IBLK), lambda b, c, s: (s[b], 0, c)),
                pl.BlockSpec((1, H, IBLK), lambda b, c, s: (s[b], 0, c)),
                pl.BlockSpec((1, IBLK, H), lambda b, c, s: (s[b], c, 0)),
            ],
            out_specs=pl.BlockSpec((B, H), lambda b, c, s: (b, 0)),
        ),
        out_shape=jax.ShapeDtypeStruct((NPAD, H), jnp.float32),
    )(smap, xs, w_gate, w_up, w_out)

    out = combine(y, pe1, po1, rp)
    return (out, hs)


# speedup vs baseline: 1.1939x; 1.1939x over previous
import functools
import jax
import jax.numpy as jnp
from jax import lax
from jax.experimental import pallas as pl
from jax.experimental.pallas import tpu as pltpu
from jax.experimental.pallas import tpu_sc as plsc

T = 2048
H = 2048
D = 256
E = 8
I = 2048

B = 512
NB = 2 * T // B + E
NPAD = NB * B
IBLK = 512
NC = 2
NS = 16
NW = NC * NS
TOK = T // NW


def _gelu_exact(x):
    return x * 0.5 * (1.0 + lax.erf(x * (2.0 ** -0.5)))


def _router_body(x_ref, wd_ref, bd_ref, rmsw_ref, w1_ref, b1_ref, w2_ref,
                 b2_ref, w3_ref, hs_ref, pe_ref, po_ref, re_ref, ro_ref,
                 bexp_ref, bval_ref):
    x = x_ref[...]
    hs = jnp.dot(x, wd_ref[...], preferred_element_type=jnp.float32)
    hs = hs + bd_ref[...][None, :]
    hs_ref[...] = hs
    ms = jnp.mean(hs * hs, axis=-1, keepdims=True)
    hsn = hs * lax.rsqrt(ms + 1e-5) * rmsw_ref[...][None, :]
    z = _gelu_exact(jnp.dot(hsn, w1_ref[...], preferred_element_type=jnp.float32)
                    + b1_ref[...][None, :])
    z = _gelu_exact(jnp.dot(z, w2_ref[...], preferred_element_type=jnp.float32)
                    + b2_ref[...][None, :])
    logits = jnp.dot(z, w3_ref[...], preferred_element_type=jnp.float32)
    m = jnp.max(logits, axis=-1, keepdims=True)
    ex = jnp.exp(logits - m)
    probs = ex / jnp.sum(ex, axis=-1, keepdims=True)
    eidx = lax.broadcasted_iota(jnp.int32, probs.shape, 1)
    m1 = jnp.max(probs, axis=-1, keepdims=True)
    i1 = jnp.min(jnp.where(probs == m1, eidx, E), axis=-1, keepdims=True)
    sel1 = eidx == i1
    masked = jnp.where(sel1, -jnp.inf, probs)
    m2 = jnp.max(masked, axis=-1, keepdims=True)
    i2 = jnp.min(jnp.where(masked == m2, eidx, E), axis=-1, keepdims=True)
    sel2 = eidx == i2
    re_ref[...] = m1
    ro_ref[...] = m2

    sel = (sel1 | sel2).astype(jnp.float32)
    r_io = lax.broadcasted_iota(jnp.int32, (T, T), 0)
    c_io = lax.broadcasted_iota(jnp.int32, (T, T), 1)
    tri = (c_io < r_io).astype(jnp.float32)
    cum_excl = jnp.dot(tri, sel, preferred_element_type=jnp.float32)
    counts = jnp.sum(sel, axis=0, keepdims=True)
    padded = jnp.ceil(counts / B) * B
    e_r = lax.broadcasted_iota(jnp.int32, (E, E), 0)
    e_c = lax.broadcasted_iota(jnp.int32, (E, E), 1)
    tri8 = (e_r <= e_c).astype(jnp.float32)
    pbase_incl = jnp.dot(padded, tri8, preferred_element_type=jnp.float32)
    pbase = pbase_incl - padded
    pos_mat = pbase + cum_excl
    pe_ref[...] = jnp.sum(jnp.where(sel1, pos_mat, 0.0), axis=-1,
                          keepdims=True).astype(jnp.int32)
    po_ref[...] = jnp.sum(jnp.where(sel2, pos_mat, 0.0), axis=-1,
                          keepdims=True).astype(jnp.int32)

    blk_start = pbase / B
    blk_cnt = padded / B
    e8 = lax.broadcasted_iota(jnp.int32, (1, E), 1).astype(jnp.float32)
    maxe = jnp.max(jnp.where(counts > 0, e8, -1.0))
    used = jnp.sum(blk_cnt)
    b_io = lax.broadcasted_iota(jnp.int32, (1, 128), 1).astype(jnp.float32)
    bexp = jnp.full((1, 128), maxe)
    for e in range(E):
        bs = jnp.sum(jnp.where(e8 == e, blk_start, 0.0))
        bc = jnp.sum(jnp.where(e8 == e, blk_cnt, 0.0))
        bexp = jnp.where((b_io >= bs) & (b_io < bs + bc), float(e), bexp)
    bexp_ref[...] = bexp.astype(jnp.int32)
    bval_ref[...] = (b_io < used).astype(jnp.int32)


def _dispatch_body(hid_ref, pe_ref, po_ref, xs_ref, pev, pov, peo, stage,
                   sem):
    wid = lax.axis_index("s") * NC + lax.axis_index("c")
    t0 = wid * TOK
    pltpu.sync_copy(pe_ref.at[pl.ds(t0, TOK)], pev)
    pltpu.sync_copy(po_ref.at[pl.ds(t0, TOK)], pov)
    nch = TOK // 16
    for c in range(nch):
        peo[c] = pev[pl.ds(16 * c, 16)]
        peo[nch + c] = pov[pl.ds(16 * c, 16)]
    for c in range(nch):
        pltpu.sync_copy(hid_ref.at[pl.ds(t0 + 16 * c, 16)], stage)
        cp1 = pltpu.async_copy(stage, xs_ref.at[peo.at[c]], sem)
        cp2 = pltpu.async_copy(stage, xs_ref.at[peo.at[nch + c]], sem)
        cp1.wait()
        cp2.wait()


def _expert_body(s_ref, x_ref, wg_ref, wu_ref, wo_ref, y_ref):
    b = pl.program_id(0)
    c = pl.program_id(1)

    @pl.when(s_ref[128 + b] == 1)
    def _():
        x = x_ref[...].astype(jnp.bfloat16)
        g = jnp.dot(x, wg_ref[0].astype(jnp.bfloat16),
                    preferred_element_type=jnp.float32)
        u = jnp.dot(x, wu_ref[0].astype(jnp.bfloat16),
                    preferred_element_type=jnp.float32)
        h = (g * jax.nn.sigmoid(g)) * u
        yc = jnp.dot(h.astype(jnp.bfloat16), wo_ref[0].astype(jnp.bfloat16),
                     preferred_element_type=jnp.float32)

        @pl.when(c == 0)
        def _():
            y_ref[...] = yc

        @pl.when(c != 0)
        def _():
            y_ref[...] += yc


_GDIMS = lax.GatherDimensionNumbers(offset_dims=(), collapsed_slice_dims=(0,),
                                    start_index_map=(0,))


def _bcast_lane(vec, lane):
    idx = jnp.full((16, 1), lane, jnp.int32)
    return lax.gather(vec, idx, _GDIMS, (1,),
                      mode=lax.GatherScatterMode.PROMISE_IN_BOUNDS)


def _combine_body(y_ref, pe_ref, po_ref, rp_ref, out_ref, pev, pov, rpv,
                  stg_e, stg_o, acc, sem):
    wid = lax.axis_index("s") * NC + lax.axis_index("c")
    t0 = wid * TOK
    pltpu.sync_copy(pe_ref.at[pl.ds(t0, TOK)], pev)
    pltpu.sync_copy(po_ref.at[pl.ds(t0, TOK)], pov)
    pltpu.sync_copy(rp_ref.at[pl.ds(2 * t0, 2 * TOK)], rpv)
    for c in range(TOK // 8):
        cpe = pltpu.async_copy(y_ref.at[pev.at[pl.ds(8 * c, 8)]], stg_e, sem)
        cpo = pltpu.async_copy(y_ref.at[pov.at[pl.ds(8 * c, 8)]], stg_o, sem)
        cpe.wait()
        cpo.wait()
        rvv = rpv[pl.ds(16 * c, 16)]
        for i in range(8):
            w0 = _bcast_lane(rvv, 2 * i)
            w1 = _bcast_lane(rvv, 2 * i + 1)

            def seg_body(j, _):
                sl = pl.ds(j * 16, 16)
                acc[i, sl] = w0 * stg_e[i, sl] + w1 * stg_o[i, sl]
                return 0

            lax.fori_loop(0, H // 16, seg_body, 0)
        pltpu.sync_copy(acc, out_ref.at[pl.ds(t0 + 8 * c, 8)])


@functools.cache
def _sc_kernels():
    mesh = plsc.VectorSubcoreMesh(core_axis_name="c", subcore_axis_name="s")
    dispatch = pl.kernel(
        _dispatch_body,
        out_type=jax.ShapeDtypeStruct((NPAD, H), jnp.float32),
        mesh=mesh,
        scratch_types=[
            pltpu.VMEM((TOK,), jnp.int32),
            pltpu.VMEM((TOK,), jnp.int32),
            pltpu.VMEM((2 * (TOK // 16), 16), jnp.int32),
            pltpu.VMEM((16, H), jnp.float32),
            pltpu.SemaphoreType.DMA,
        ],
    )
    combine = pl.kernel(
        _combine_body,
        out_type=jax.ShapeDtypeStruct((T, H), jnp.float32),
        mesh=mesh,
        scratch_types=[
            pltpu.VMEM((TOK,), jnp.int32),
            pltpu.VMEM((TOK,), jnp.int32),
            pltpu.VMEM((2 * TOK,), jnp.float32),
            pltpu.VMEM((8, H), jnp.float32),
            pltpu.VMEM((8, H), jnp.float32),
            pltpu.VMEM((8, H), jnp.float32),
            pltpu.SemaphoreType.DMA,
        ],
    )
    return dispatch, combine


@jax.jit
def kernel(hidden_states, W_down, b_down, rms_w, W_r1, b_r1, W_r2, b_r2,
           W_r3, w_gate, w_up, w_out):
    hs, pe, po, re, ro, bexp, bval = pl.pallas_call(
        _router_body,
        out_shape=(
            jax.ShapeDtypeStruct((T, D), jnp.float32),
            jax.ShapeDtypeStruct((T, 1), jnp.int32),
            jax.ShapeDtypeStruct((T, 1), jnp.int32),
            jax.ShapeDtypeStruct((T, 1), jnp.float32),
            jax.ShapeDtypeStruct((T, 1), jnp.float32),
            jax.ShapeDtypeStruct((1, 128), jnp.int32),
            jax.ShapeDtypeStruct((1, 128), jnp.int32),
        ),
    )(hidden_states, W_down, b_down, rms_w, W_r1, b_r1, W_r2, b_r2, W_r3)

    pe1 = pe.reshape(T)
    po1 = po.reshape(T)
    rp = jnp.concatenate([re, ro], axis=1).reshape(2 * T)
    smap = jnp.concatenate([bexp, bval], axis=0).reshape(256)

    dispatch, combine = _sc_kernels()
    xs = dispatch(hidden_states, pe1, po1)

    y = pl.pallas_call(
        _expert_body,
        grid_spec=pltpu.PrefetchScalarGridSpec(
            num_scalar_prefetch=1,
            grid=(NB, I // IBLK),
            in_specs=[
                pl.BlockSpec((B, H), lambda b, c, s: (b, 0)),
                pl.BlockSpec((1, H, IBLK), lambda b, c, s: (s[b], 0, c)),
                pl.BlockSpec((1, H, IBLK), lambda b, c, s: (s[b], 0, c)),
                pl.BlockSpec((1, IBLK, H), lambda b, c, s: (s[b], c, 0)),
            ],
            out_specs=pl.BlockSpec((B, H), lambda b, c, s: (b, 0)),
        ),
        out_shape=jax.ShapeDtypeStruct((NPAD, H), jnp.float32),
    )(smap, xs, w_gate, w_up, w_out)

    out = combine(y, pe1, po1, rp)
    return (out, hs)


# speedup vs baseline: 1.2390x; 1.0378x over previous
import functools
import jax
import jax.numpy as jnp
from jax import lax
from jax.experimental import pallas as pl
from jax.experimental.pallas import tpu as pltpu
from jax.experimental.pallas import tpu_sc as plsc

T = 2048
H = 2048
D = 256
E = 8
I = 2048

B = 512
NB = 2 * T // B + E
NPAD = NB * B
IBLK = 512
NC = 2
NS = 16
NW = NC * NS
TOK = T // NW


def _gelu_exact(x):
    return x * 0.5 * (1.0 + lax.erf(x * (2.0 ** -0.5)))


def _router_body(x_ref, wd_ref, bd_ref, rmsw_ref, w1_ref, b1_ref, w2_ref,
                 b2_ref, w3_ref, hs_ref, pe_ref, po_ref, re_ref, ro_ref,
                 bexp_ref, bval_ref):
    x = x_ref[...]
    hs = jnp.dot(x, wd_ref[...], preferred_element_type=jnp.float32)
    hs = hs + bd_ref[...][None, :]
    hs_ref[...] = hs
    ms = jnp.mean(hs * hs, axis=-1, keepdims=True)
    hsn = hs * lax.rsqrt(ms + 1e-5) * rmsw_ref[...][None, :]
    z = _gelu_exact(jnp.dot(hsn, w1_ref[...], preferred_element_type=jnp.float32)
                    + b1_ref[...][None, :])
    z = _gelu_exact(jnp.dot(z, w2_ref[...], preferred_element_type=jnp.float32)
                    + b2_ref[...][None, :])
    logits = jnp.dot(z, w3_ref[...], preferred_element_type=jnp.float32)
    m = jnp.max(logits, axis=-1, keepdims=True)
    ex = jnp.exp(logits - m)
    probs = ex / jnp.sum(ex, axis=-1, keepdims=True)
    eidx = lax.broadcasted_iota(jnp.int32, probs.shape, 1)
    m1 = jnp.max(probs, axis=-1, keepdims=True)
    i1 = jnp.min(jnp.where(probs == m1, eidx, E), axis=-1, keepdims=True)
    sel1 = eidx == i1
    masked = jnp.where(sel1, -jnp.inf, probs)
    m2 = jnp.max(masked, axis=-1, keepdims=True)
    i2 = jnp.min(jnp.where(masked == m2, eidx, E), axis=-1, keepdims=True)
    sel2 = eidx == i2
    re_ref[...] = m1
    ro_ref[...] = m2

    sel = (sel1 | sel2).astype(jnp.float32)
    r_io = lax.broadcasted_iota(jnp.int32, (T, T), 0)
    c_io = lax.broadcasted_iota(jnp.int32, (T, T), 1)
    tri = (c_io < r_io).astype(jnp.float32)
    cum_excl = jnp.dot(tri, sel, preferred_element_type=jnp.float32)
    counts = jnp.sum(sel, axis=0, keepdims=True)
    padded = jnp.ceil(counts / B) * B
    e_r = lax.broadcasted_iota(jnp.int32, (E, E), 0)
    e_c = lax.broadcasted_iota(jnp.int32, (E, E), 1)
    tri8 = (e_r <= e_c).astype(jnp.float32)
    pbase_incl = jnp.dot(padded, tri8, preferred_element_type=jnp.float32)
    pbase = pbase_incl - padded
    pos_mat = pbase + cum_excl
    pe_ref[...] = jnp.sum(jnp.where(sel1, pos_mat, 0.0), axis=-1,
                          keepdims=True).astype(jnp.int32)
    po_ref[...] = jnp.sum(jnp.where(sel2, pos_mat, 0.0), axis=-1,
                          keepdims=True).astype(jnp.int32)

    blk_start = pbase / B
    blk_cnt = padded / B
    e8 = lax.broadcasted_iota(jnp.int32, (1, E), 1).astype(jnp.float32)
    maxe = jnp.max(jnp.where(counts > 0, e8, -1.0))
    used = jnp.sum(blk_cnt)
    b_io = lax.broadcasted_iota(jnp.int32, (1, 128), 1).astype(jnp.float32)
    bexp = jnp.full((1, 128), maxe)
    for e in range(E):
        bs = jnp.sum(jnp.where(e8 == e, blk_start, 0.0))
        bc = jnp.sum(jnp.where(e8 == e, blk_cnt, 0.0))
        bexp = jnp.where((b_io >= bs) & (b_io < bs + bc), float(e), bexp)
    bexp_ref[...] = bexp.astype(jnp.int32)
    bval_ref[...] = (b_io < used).astype(jnp.int32)


def _dispatch_body(hid_ref, pe_ref, po_ref, xs_ref, pev, pov, peo, stage,
                   sem):
    wid = lax.axis_index("s") * NC + lax.axis_index("c")
    t0 = wid * TOK
    pltpu.sync_copy(pe_ref.at[pl.ds(t0, TOK)], pev)
    pltpu.sync_copy(po_ref.at[pl.ds(t0, TOK)], pov)
    nch = TOK // 16
    for c in range(nch):
        peo[c] = pev[pl.ds(16 * c, 16)]
        peo[nch + c] = pov[pl.ds(16 * c, 16)]
    for c in range(nch):
        pltpu.sync_copy(hid_ref.at[pl.ds(t0 + 16 * c, 16)], stage)
        cp1 = pltpu.async_copy(stage, xs_ref.at[peo.at[c]], sem)
        cp2 = pltpu.async_copy(stage, xs_ref.at[peo.at[nch + c]], sem)
        cp1.wait()
        cp2.wait()


def _expert_body(s_ref, x_ref, wg_ref, wu_ref, wo_ref, y_ref):
    b = pl.program_id(0)
    c = pl.program_id(1)

    @pl.when(s_ref[128 + b] == 1)
    def _():
        x = x_ref[...].astype(jnp.bfloat16)
        g = jnp.dot(x, wg_ref[0].astype(jnp.bfloat16),
                    preferred_element_type=jnp.float32)
        u = jnp.dot(x, wu_ref[0].astype(jnp.bfloat16),
                    preferred_element_type=jnp.float32)
        h = (g * jax.nn.sigmoid(g)) * u
        yc = jnp.dot(h.astype(jnp.bfloat16), wo_ref[0].astype(jnp.bfloat16),
                     preferred_element_type=jnp.float32)

        @pl.when(c == 0)
        def _():
            y_ref[...] = yc

        @pl.when(c != 0)
        def _():
            y_ref[...] += yc


_GDIMS = lax.GatherDimensionNumbers(offset_dims=(), collapsed_slice_dims=(0,),
                                    start_index_map=(0,))


def _bcast_lane(vec, lane):
    idx = jnp.full((16, 1), lane, jnp.int32)
    return lax.gather(vec, idx, _GDIMS, (1,),
                      mode=lax.GatherScatterMode.PROMISE_IN_BOUNDS)


def _combine_body(y_ref, pe_ref, po_ref, rp_ref, out_ref, pev, pov, rpv,
                  stg_e, stg_o, acc, sem):
    wid = lax.axis_index("s") * NC + lax.axis_index("c")
    t0 = wid * TOK
    pltpu.sync_copy(pe_ref.at[pl.ds(t0, TOK)], pev)
    pltpu.sync_copy(po_ref.at[pl.ds(t0, TOK)], pov)
    pltpu.sync_copy(rp_ref.at[pl.ds(2 * t0, 2 * TOK)], rpv)
    for c in range(TOK // 8):
        cpe = pltpu.async_copy(y_ref.at[pev.at[pl.ds(8 * c, 8)]], stg_e, sem)
        cpo = pltpu.async_copy(y_ref.at[pov.at[pl.ds(8 * c, 8)]], stg_o, sem)
        cpe.wait()
        cpo.wait()
        rvv = rpv[pl.ds(16 * c, 16)]
        for i in range(8):
            w0 = _bcast_lane(rvv, 2 * i)
            w1 = _bcast_lane(rvv, 2 * i + 1)

            def seg_body(j, _):
                for u in range(4):
                    sl = pl.ds(j * 64 + u * 16, 16)
                    acc[i, sl] = w0 * stg_e[i, sl] + w1 * stg_o[i, sl]
                return 0

            lax.fori_loop(0, H // 64, seg_body, 0)
        pltpu.sync_copy(acc, out_ref.at[pl.ds(t0 + 8 * c, 8)])


_NCH = I // IBLK


def _serp(b, c):
    return jnp.where(b % 2 == 1, _NCH - 1 - c, c)

@functools.cache
def _sc_kernels():
    mesh = plsc.VectorSubcoreMesh(core_axis_name="c", subcore_axis_name="s")
    dispatch = pl.kernel(
        _dispatch_body,
        out_type=jax.ShapeDtypeStruct((NPAD, H), jnp.float32),
        mesh=mesh,
        scratch_types=[
            pltpu.VMEM((TOK,), jnp.int32),
            pltpu.VMEM((TOK,), jnp.int32),
            pltpu.VMEM((2 * (TOK // 16), 16), jnp.int32),
            pltpu.VMEM((16, H), jnp.float32),
            pltpu.SemaphoreType.DMA,
        ],
    )
    combine = pl.kernel(
        _combine_body,
        out_type=jax.ShapeDtypeStruct((T, H), jnp.float32),
        mesh=mesh,
        scratch_types=[
            pltpu.VMEM((TOK,), jnp.int32),
            pltpu.VMEM((TOK,), jnp.int32),
            pltpu.VMEM((2 * TOK,), jnp.float32),
            pltpu.VMEM((8, H), jnp.float32),
            pltpu.VMEM((8, H), jnp.float32),
            pltpu.VMEM((8, H), jnp.float32),
            pltpu.SemaphoreType.DMA,
        ],
    )
    return dispatch, combine


@jax.jit
def kernel(hidden_states, W_down, b_down, rms_w, W_r1, b_r1, W_r2, b_r2,
           W_r3, w_gate, w_up, w_out):
    hs, pe, po, re, ro, bexp, bval = pl.pallas_call(
        _router_body,
        out_shape=(
            jax.ShapeDtypeStruct((T, D), jnp.float32),
            jax.ShapeDtypeStruct((T, 1), jnp.int32),
            jax.ShapeDtypeStruct((T, 1), jnp.int32),
            jax.ShapeDtypeStruct((T, 1), jnp.float32),
            jax.ShapeDtypeStruct((T, 1), jnp.float32),
            jax.ShapeDtypeStruct((1, 128), jnp.int32),
            jax.ShapeDtypeStruct((1, 128), jnp.int32),
        ),
    )(hidden_states, W_down, b_down, rms_w, W_r1, b_r1, W_r2, b_r2, W_r3)

    pe1 = pe.reshape(T)
    po1 = po.reshape(T)
    rp = jnp.concatenate([re, ro], axis=1).reshape(2 * T)
    smap = jnp.concatenate([bexp, bval], axis=0).reshape(256)

    dispatch, combine = _sc_kernels()
    xs = dispatch(hidden_states, pe1, po1)

    y = pl.pallas_call(
        _expert_body,
        grid_spec=pltpu.PrefetchScalarGridSpec(
            num_scalar_prefetch=1,
            grid=(NB, I // IBLK),
            in_specs=[
                pl.BlockSpec((B, H), lambda b, c, s: (b, 0)),
                pl.BlockSpec((1, H, IBLK), lambda b, c, s: (s[b], 0, _serp(b, c))),
                pl.BlockSpec((1, H, IBLK), lambda b, c, s: (s[b], 0, _serp(b, c))),
                pl.BlockSpec((1, IBLK, H), lambda b, c, s: (s[b], _serp(b, c), 0)),
            ],
            out_specs=pl.BlockSpec((B, H), lambda b, c, s: (b, 0)),
        ),
        out_shape=jax.ShapeDtypeStruct((NPAD, H), jnp.float32),
    )(smap, xs, w_gate, w_up, w_out)

    out = combine(y, pe1, po1, rp)
    return (out, hs)


# speedup vs baseline: 1.4441x; 1.1655x over previous
import functools
import jax
import jax.numpy as jnp
from jax import lax
from jax.experimental import pallas as pl
from jax.experimental.pallas import tpu as pltpu
from jax.experimental.pallas import tpu_sc as plsc

T = 2048
H = 2048
D = 256
E = 8
I = 2048

B = 512
NB = 2 * T // B + E
NPAD = NB * B
IBLK = 512
NC = 2
NS = 16
NW = NC * NS
TOK = T // NW


def _gelu_exact(x):
    return x * 0.5 * (1.0 + lax.erf(x * (2.0 ** -0.5)))


def _router_body(x_ref, wd_ref, bd_ref, rmsw_ref, w1_ref, b1_ref, w2_ref,
                 b2_ref, w3_ref, hs_ref, pe_ref, po_ref, re_ref, ro_ref,
                 bexp_ref, bval_ref):
    x = x_ref[...]
    hs = jnp.dot(x, wd_ref[...], preferred_element_type=jnp.float32)
    hs = hs + bd_ref[...][None, :]
    hs_ref[...] = hs
    ms = jnp.mean(hs * hs, axis=-1, keepdims=True)
    hsn = hs * lax.rsqrt(ms + 1e-5) * rmsw_ref[...][None, :]
    z = _gelu_exact(jnp.dot(hsn, w1_ref[...], preferred_element_type=jnp.float32)
                    + b1_ref[...][None, :])
    z = _gelu_exact(jnp.dot(z, w2_ref[...], preferred_element_type=jnp.float32)
                    + b2_ref[...][None, :])
    logits = jnp.dot(z, w3_ref[...], preferred_element_type=jnp.float32)
    m = jnp.max(logits, axis=-1, keepdims=True)
    ex = jnp.exp(logits - m)
    probs = ex / jnp.sum(ex, axis=-1, keepdims=True)
    eidx = lax.broadcasted_iota(jnp.int32, probs.shape, 1)
    m1 = jnp.max(probs, axis=-1, keepdims=True)
    i1 = jnp.min(jnp.where(probs == m1, eidx, E), axis=-1, keepdims=True)
    sel1 = eidx == i1
    masked = jnp.where(sel1, -jnp.inf, probs)
    m2 = jnp.max(masked, axis=-1, keepdims=True)
    i2 = jnp.min(jnp.where(masked == m2, eidx, E), axis=-1, keepdims=True)
    sel2 = eidx == i2
    re_ref[...] = m1
    ro_ref[...] = m2

    sel = (sel1 | sel2).astype(jnp.float32)
    r_io = lax.broadcasted_iota(jnp.int32, (T, T), 0)
    c_io = lax.broadcasted_iota(jnp.int32, (T, T), 1)
    tri = (c_io < r_io).astype(jnp.float32)
    cum_excl = jnp.dot(tri, sel, preferred_element_type=jnp.float32)
    counts = jnp.sum(sel, axis=0, keepdims=True)
    padded = jnp.ceil(counts / B) * B
    e_r = lax.broadcasted_iota(jnp.int32, (E, E), 0)
    e_c = lax.broadcasted_iota(jnp.int32, (E, E), 1)
    tri8 = (e_r <= e_c).astype(jnp.float32)
    pbase_incl = jnp.dot(padded, tri8, preferred_element_type=jnp.float32)
    pbase = pbase_incl - padded
    pos_mat = pbase + cum_excl
    pe_ref[...] = jnp.sum(jnp.where(sel1, pos_mat, 0.0), axis=-1,
                          keepdims=True).astype(jnp.int32)
    po_ref[...] = jnp.sum(jnp.where(sel2, pos_mat, 0.0), axis=-1,
                          keepdims=True).astype(jnp.int32)

    blk_start = pbase / B
    blk_cnt = padded / B
    e8 = lax.broadcasted_iota(jnp.int32, (1, E), 1).astype(jnp.float32)
    maxe = jnp.max(jnp.where(counts > 0, e8, -1.0))
    used = jnp.sum(blk_cnt)
    b_io = lax.broadcasted_iota(jnp.int32, (1, 128), 1).astype(jnp.float32)
    bexp = jnp.full((1, 128), maxe)
    for e in range(E):
        bs = jnp.sum(jnp.where(e8 == e, blk_start, 0.0))
        bc = jnp.sum(jnp.where(e8 == e, blk_cnt, 0.0))
        bexp = jnp.where((b_io >= bs) & (b_io < bs + bc), float(e), bexp)
    bexp_ref[...] = bexp.astype(jnp.int32)
    u1 = used - 1.0
    par = u1 - 2.0 * jnp.floor(u1 * 0.5)
    cend = jnp.where(par > 0.5, 0.0, float(I // IBLK - 1))
    bval = (b_io < used).astype(jnp.float32)
    bval = jnp.where(b_io == 126.0, u1, bval)
    bval = jnp.where(b_io == 127.0, cend, bval)
    bval_ref[...] = bval.astype(jnp.int32)


def _dispatch_body(hid_ref, pe_ref, po_ref, xs_ref, pev, pov, peo, stage,
                   sem):
    wid = lax.axis_index("s") * NC + lax.axis_index("c")
    t0 = wid * TOK
    pltpu.sync_copy(pe_ref.at[pl.ds(t0, TOK)], pev)
    pltpu.sync_copy(po_ref.at[pl.ds(t0, TOK)], pov)
    nch = TOK // 16
    for c in range(nch):
        peo[c] = pev[pl.ds(16 * c, 16)]
        peo[nch + c] = pov[pl.ds(16 * c, 16)]
    for c in range(nch):
        pltpu.sync_copy(hid_ref.at[pl.ds(t0 + 16 * c, 16)], stage)
        cp1 = pltpu.async_copy(stage, xs_ref.at[peo.at[c]], sem)
        cp2 = pltpu.async_copy(stage, xs_ref.at[peo.at[nch + c]], sem)
        cp1.wait()
        cp2.wait()


def _expert_body(s_ref, x_ref, wg_ref, wu_ref, wo_ref, y_ref):
    b = pl.program_id(0)
    c = pl.program_id(1)

    @pl.when(s_ref[128 + b] == 1)
    def _():
        x = x_ref[...].astype(jnp.bfloat16)
        g = jnp.dot(x, wg_ref[0].astype(jnp.bfloat16),
                    preferred_element_type=jnp.float32)
        u = jnp.dot(x, wu_ref[0].astype(jnp.bfloat16),
                    preferred_element_type=jnp.float32)
        h = (g * jax.nn.sigmoid(g)) * u
        yc = jnp.dot(h.astype(jnp.bfloat16), wo_ref[0].astype(jnp.bfloat16),
                     preferred_element_type=jnp.float32)

        @pl.when(c == 0)
        def _():
            y_ref[...] = yc

        @pl.when(c != 0)
        def _():
            y_ref[...] += yc


_GDIMS = lax.GatherDimensionNumbers(offset_dims=(), collapsed_slice_dims=(0,),
                                    start_index_map=(0,))


def _bcast_lane(vec, lane):
    idx = jnp.full((16, 1), lane, jnp.int32)
    return lax.gather(vec, idx, _GDIMS, (1,),
                      mode=lax.GatherScatterMode.PROMISE_IN_BOUNDS)


def _combine_body(y_ref, pe_ref, po_ref, rp_ref, out_ref, pev, pov, rpv,
                  stg_e, stg_o, acc, sem):
    wid = lax.axis_index("s") * NC + lax.axis_index("c")
    t0 = wid * TOK
    pltpu.sync_copy(pe_ref.at[pl.ds(t0, TOK)], pev)
    pltpu.sync_copy(po_ref.at[pl.ds(t0, TOK)], pov)
    pltpu.sync_copy(rp_ref.at[pl.ds(2 * t0, 2 * TOK)], rpv)
    for c in range(TOK // 8):
        cpe = pltpu.async_copy(y_ref.at[pev.at[pl.ds(8 * c, 8)]], stg_e, sem)
        cpo = pltpu.async_copy(y_ref.at[pov.at[pl.ds(8 * c, 8)]], stg_o, sem)
        cpe.wait()
        cpo.wait()
        rvv = rpv[pl.ds(16 * c, 16)]
        for i in range(8):
            w0 = _bcast_lane(rvv, 2 * i)
            w1 = _bcast_lane(rvv, 2 * i + 1)

            def seg_body(j, _):
                for u in range(4):
                    sl = pl.ds(j * 64 + u * 16, 16)
                    acc[i, sl] = w0 * stg_e[i, sl] + w1 * stg_o[i, sl]
                return 0

            lax.fori_loop(0, H // 64, seg_body, 0)
        pltpu.sync_copy(acc, out_ref.at[pl.ds(t0 + 8 * c, 8)])


_NCH = I // IBLK


def _serp(b, c):
    return jnp.where(b % 2 == 1, _NCH - 1 - c, c)


def _cidx(b, c, s):
    return jnp.where(s[128 + b] == 1, _serp(b, c), s[255])


def _xidx(b, s):
    return jnp.where(s[128 + b] == 1, b, s[254])

@functools.cache
def _sc_kernels():
    mesh = plsc.VectorSubcoreMesh(core_axis_name="c", subcore_axis_name="s")
    dispatch = pl.kernel(
        _dispatch_body,
        out_type=jax.ShapeDtypeStruct((NPAD, H), jnp.float32),
        mesh=mesh,
        scratch_types=[
            pltpu.VMEM((TOK,), jnp.int32),
            pltpu.VMEM((TOK,), jnp.int32),
            pltpu.VMEM((2 * (TOK // 16), 16), jnp.int32),
            pltpu.VMEM((16, H), jnp.float32),
            pltpu.SemaphoreType.DMA,
        ],
    )
    combine = pl.kernel(
        _combine_body,
        out_type=jax.ShapeDtypeStruct((T, H), jnp.float32),
        mesh=mesh,
        scratch_types=[
            pltpu.VMEM((TOK,), jnp.int32),
            pltpu.VMEM((TOK,), jnp.int32),
            pltpu.VMEM((2 * TOK,), jnp.float32),
            pltpu.VMEM((8, H), jnp.float32),
            pltpu.VMEM((8, H), jnp.float32),
            pltpu.VMEM((8, H), jnp.float32),
            pltpu.SemaphoreType.DMA,
        ],
    )
    return dispatch, combine


@jax.jit
def kernel(hidden_states, W_down, b_down, rms_w, W_r1, b_r1, W_r2, b_r2,
           W_r3, w_gate, w_up, w_out):
    hs, pe, po, re, ro, bexp, bval = pl.pallas_call(
        _router_body,
        out_shape=(
            jax.ShapeDtypeStruct((T, D), jnp.float32),
            jax.ShapeDtypeStruct((T, 1), jnp.int32),
            jax.ShapeDtypeStruct((T, 1), jnp.int32),
            jax.ShapeDtypeStruct((T, 1), jnp.float32),
            jax.ShapeDtypeStruct((T, 1), jnp.float32),
            jax.ShapeDtypeStruct((1, 128), jnp.int32),
            jax.ShapeDtypeStruct((1, 128), jnp.int32),
        ),
    )(hidden_states, W_down, b_down, rms_w, W_r1, b_r1, W_r2, b_r2, W_r3)

    pe1 = pe.reshape(T)
    po1 = po.reshape(T)
    rp = jnp.concatenate([re, ro], axis=1).reshape(2 * T)
    smap = jnp.concatenate([bexp, bval], axis=0).reshape(256)

    dispatch, combine = _sc_kernels()
    xs = dispatch(hidden_states, pe1, po1)

    y = pl.pallas_call(
        _expert_body,
        grid_spec=pltpu.PrefetchScalarGridSpec(
            num_scalar_prefetch=1,
            grid=(NB, I // IBLK),
            in_specs=[
                pl.BlockSpec((B, H), lambda b, c, s: (_xidx(b, s), 0)),
                pl.BlockSpec((1, H, IBLK), lambda b, c, s: (s[b], 0, _cidx(b, c, s))),
                pl.BlockSpec((1, H, IBLK), lambda b, c, s: (s[b], 0, _cidx(b, c, s))),
                pl.BlockSpec((1, IBLK, H), lambda b, c, s: (s[b], _cidx(b, c, s), 0)),
            ],
            out_specs=pl.BlockSpec((B, H), lambda b, c, s: (b, 0)),
        ),
        out_shape=jax.ShapeDtypeStruct((NPAD, H), jnp.float32),
    )(smap, xs, w_gate, w_up, w_out)

    out = combine(y, pe1, po1, rp)
    return (out, hs)


# speedup vs baseline: 1.4594x; 1.0106x over previous
import functools
import jax
import jax.numpy as jnp
from jax import lax
from jax.experimental import pallas as pl
from jax.experimental.pallas import tpu as pltpu
from jax.experimental.pallas import tpu_sc as plsc

T = 2048
H = 2048
D = 256
E = 8
I = 2048

B = 512
NB = 2 * T // B + E
NPAD = NB * B
IBLK = 512
NC = 2
NS = 16
NW = NC * NS
TOK = T // NW


def _gelu_exact(x):
    return x * 0.5 * (1.0 + lax.erf(x * (2.0 ** -0.5)))


def _router_body(x_ref, wd_ref, bd_ref, rmsw_ref, w1_ref, b1_ref, w2_ref,
                 b2_ref, w3_ref, hs_ref, pe_ref, po_ref, re_ref, ro_ref,
                 bexp_ref, bval_ref):
    x = x_ref[...]
    hs = jnp.dot(x, wd_ref[...], preferred_element_type=jnp.float32)
    hs = hs + bd_ref[...][None, :]
    hs_ref[...] = hs
    ms = jnp.mean(hs * hs, axis=-1, keepdims=True)
    hsn = hs * lax.rsqrt(ms + 1e-5) * rmsw_ref[...][None, :]
    z = _gelu_exact(jnp.dot(hsn, w1_ref[...], preferred_element_type=jnp.float32)
                    + b1_ref[...][None, :])
    z = _gelu_exact(jnp.dot(z, w2_ref[...], preferred_element_type=jnp.float32)
                    + b2_ref[...][None, :])
    logits = jnp.dot(z, w3_ref[...], preferred_element_type=jnp.float32)
    m = jnp.max(logits, axis=-1, keepdims=True)
    ex = jnp.exp(logits - m)
    probs = ex / jnp.sum(ex, axis=-1, keepdims=True)
    eidx = lax.broadcasted_iota(jnp.int32, probs.shape, 1)
    m1 = jnp.max(probs, axis=-1, keepdims=True)
    i1 = jnp.min(jnp.where(probs == m1, eidx, E), axis=-1, keepdims=True)
    sel1 = eidx == i1
    masked = jnp.where(sel1, -jnp.inf, probs)
    m2 = jnp.max(masked, axis=-1, keepdims=True)
    i2 = jnp.min(jnp.where(masked == m2, eidx, E), axis=-1, keepdims=True)
    sel2 = eidx == i2
    re_ref[...] = m1
    ro_ref[...] = m2

    sel = (sel1 | sel2).astype(jnp.float32)
    r_io = lax.broadcasted_iota(jnp.int32, (T, T), 0)
    c_io = lax.broadcasted_iota(jnp.int32, (T, T), 1)
    tri = (c_io < r_io).astype(jnp.float32)
    cum_excl = jnp.dot(tri, sel, preferred_element_type=jnp.float32)
    counts = jnp.sum(sel, axis=0, keepdims=True)
    padded = jnp.ceil(counts / B) * B
    e_r = lax.broadcasted_iota(jnp.int32, (E, E), 0)
    e_c = lax.broadcasted_iota(jnp.int32, (E, E), 1)
    tri8 = (e_r <= e_c).astype(jnp.float32)
    pbase_incl = jnp.dot(padded, tri8, preferred_element_type=jnp.float32)
    pbase = pbase_incl - padded
    pos_mat = pbase + cum_excl
    pe_ref[...] = jnp.sum(jnp.where(sel1, pos_mat, 0.0), axis=-1,
                          keepdims=True).astype(jnp.int32)
    po_ref[...] = jnp.sum(jnp.where(sel2, pos_mat, 0.0), axis=-1,
                          keepdims=True).astype(jnp.int32)

    blk_start = pbase / B
    blk_cnt = padded / B
    e8 = lax.broadcasted_iota(jnp.int32, (1, E), 1).astype(jnp.float32)
    maxe = jnp.max(jnp.where(counts > 0, e8, -1.0))
    used = jnp.sum(blk_cnt)
    b_io = lax.broadcasted_iota(jnp.int32, (1, 128), 1).astype(jnp.float32)
    bexp = jnp.full((1, 128), maxe)
    for e in range(E):
        bs = jnp.sum(jnp.where(e8 == e, blk_start, 0.0))
        bc = jnp.sum(jnp.where(e8 == e, blk_cnt, 0.0))
        bexp = jnp.where((b_io >= bs) & (b_io < bs + bc), float(e), bexp)
    bexp_ref[...] = bexp.astype(jnp.int32)
    u1 = used - 1.0
    par = u1 - 2.0 * jnp.floor(u1 * 0.5)
    cend = jnp.where(par > 0.5, 0.0, float(I // IBLK - 1))
    bval = (b_io < used).astype(jnp.float32)
    bval = jnp.where(b_io == 126.0, u1, bval)
    bval = jnp.where(b_io == 127.0, cend, bval)
    bval_ref[...] = bval.astype(jnp.int32)


def _dispatch_body(hid_ref, pe_ref, po_ref, xs_ref, pev, pov, peo, stage,
                   sem):
    wid = lax.axis_index("s") * NC + lax.axis_index("c")
    t0 = wid * TOK
    pltpu.sync_copy(pe_ref.at[pl.ds(t0, TOK)], pev)
    pltpu.sync_copy(po_ref.at[pl.ds(t0, TOK)], pov)
    nch = TOK // 16
    for c in range(nch):
        peo[c] = pev[pl.ds(16 * c, 16)]
        peo[nch + c] = pov[pl.ds(16 * c, 16)]
    for c in range(nch):
        pltpu.sync_copy(hid_ref.at[pl.ds(t0 + 16 * c, 16)], stage)
        cp1 = pltpu.async_copy(stage, xs_ref.at[peo.at[c]], sem)
        cp2 = pltpu.async_copy(stage, xs_ref.at[peo.at[nch + c]], sem)
        cp1.wait()
        cp2.wait()


def _expert_body(s_ref, x_ref, wg_ref, wu_ref, wo_ref, y_ref):
    b = pl.program_id(0)
    c = pl.program_id(1)

    @pl.when(s_ref[128 + b] == 1)
    def _():
        x = x_ref[...].astype(jnp.bfloat16)
        g = jnp.dot(x, wg_ref[0].astype(jnp.bfloat16),
                    preferred_element_type=jnp.float32)
        u = jnp.dot(x, wu_ref[0].astype(jnp.bfloat16),
                    preferred_element_type=jnp.float32)
        h = (g * jax.nn.sigmoid(g)) * u
        yc = jnp.dot(h.astype(jnp.bfloat16), wo_ref[0].astype(jnp.bfloat16),
                     preferred_element_type=jnp.float32)

        @pl.when(c == 0)
        def _():
            y_ref[...] = yc

        @pl.when(c != 0)
        def _():
            y_ref[...] += yc


_GDIMS = lax.GatherDimensionNumbers(offset_dims=(), collapsed_slice_dims=(0,),
                                    start_index_map=(0,))


def _bcast_lane(vec, lane):
    idx = jnp.full((16, 1), lane, jnp.int32)
    return lax.gather(vec, idx, _GDIMS, (1,),
                      mode=lax.GatherScatterMode.PROMISE_IN_BOUNDS)


def _combine_body(y_ref, pe_ref, po_ref, rp_ref, out_ref, pev, pov, rpv,
                  stg_e, stg_o, acc, sem):
    wid = lax.axis_index("s") * NC + lax.axis_index("c")
    t0 = wid * TOK
    pltpu.sync_copy(pe_ref.at[pl.ds(t0, TOK)], pev)
    pltpu.sync_copy(po_ref.at[pl.ds(t0, TOK)], pov)
    pltpu.sync_copy(rp_ref.at[pl.ds(2 * t0, 2 * TOK)], rpv)
    nch = TOK // 8

    def issue(c, buf):
        cpe = pltpu.async_copy(y_ref.at[pev.at[pl.ds(8 * c, 8)]],
                               stg_e.at[buf], sem)
        cpo = pltpu.async_copy(y_ref.at[pov.at[pl.ds(8 * c, 8)]],
                               stg_o.at[buf], sem)
        return cpe, cpo

    pend = issue(0, 0)
    for c in range(nch):
        buf = c % 2
        cur = pend
        if c + 1 < nch:
            pend = issue(c + 1, 1 - buf)
        cur[0].wait()
        cur[1].wait()
        rvv = rpv[pl.ds(16 * c, 16)]
        for i in range(8):
            w0 = _bcast_lane(rvv, 2 * i)
            w1 = _bcast_lane(rvv, 2 * i + 1)

            def seg_body(j, _):
                for u in range(4):
                    sl = pl.ds(j * 64 + u * 16, 16)
                    acc[i, sl] = (w0 * stg_e[buf, i, sl]
                                  + w1 * stg_o[buf, i, sl])
                return 0

            lax.fori_loop(0, H // 64, seg_body, 0)
        pltpu.sync_copy(acc, out_ref.at[pl.ds(t0 + 8 * c, 8)])


_NCH = I // IBLK


def _serp(b, c):
    return jnp.where(b % 2 == 1, _NCH - 1 - c, c)


def _cidx(b, c, s):
    return jnp.where(s[128 + b] == 1, _serp(b, c), s[255])


def _xidx(b, s):
    return jnp.where(s[128 + b] == 1, b, s[254])

@functools.cache
def _sc_kernels():
    mesh = plsc.VectorSubcoreMesh(core_axis_name="c", subcore_axis_name="s")
    dispatch = pl.kernel(
        _dispatch_body,
        out_type=jax.ShapeDtypeStruct((NPAD, H), jnp.float32),
        mesh=mesh,
        scratch_types=[
            pltpu.VMEM((TOK,), jnp.int32),
            pltpu.VMEM((TOK,), jnp.int32),
            pltpu.VMEM((2 * (TOK // 16), 16), jnp.int32),
            pltpu.VMEM((16, H), jnp.float32),
            pltpu.SemaphoreType.DMA,
        ],
    )
    combine = pl.kernel(
        _combine_body,
        out_type=jax.ShapeDtypeStruct((T, H), jnp.float32),
        mesh=mesh,
        scratch_types=[
            pltpu.VMEM((TOK,), jnp.int32),
            pltpu.VMEM((TOK,), jnp.int32),
            pltpu.VMEM((2 * TOK,), jnp.float32),
            pltpu.VMEM((2, 8, H), jnp.float32),
            pltpu.VMEM((2, 8, H), jnp.float32),
            pltpu.VMEM((8, H), jnp.float32),
            pltpu.SemaphoreType.DMA,
        ],
    )
    return dispatch, combine


@jax.jit
def kernel(hidden_states, W_down, b_down, rms_w, W_r1, b_r1, W_r2, b_r2,
           W_r3, w_gate, w_up, w_out):
    hs, pe, po, re, ro, bexp, bval = pl.pallas_call(
        _router_body,
        out_shape=(
            jax.ShapeDtypeStruct((T, D), jnp.float32),
            jax.ShapeDtypeStruct((T, 1), jnp.int32),
            jax.ShapeDtypeStruct((T, 1), jnp.int32),
            jax.ShapeDtypeStruct((T, 1), jnp.float32),
            jax.ShapeDtypeStruct((T, 1), jnp.float32),
            jax.ShapeDtypeStruct((1, 128), jnp.int32),
            jax.ShapeDtypeStruct((1, 128), jnp.int32),
        ),
    )(hidden_states, W_down, b_down, rms_w, W_r1, b_r1, W_r2, b_r2, W_r3)

    pe1 = pe.reshape(T)
    po1 = po.reshape(T)
    rp = jnp.concatenate([re, ro], axis=1).reshape(2 * T)
    smap = jnp.concatenate([bexp, bval], axis=0).reshape(256)

    dispatch, combine = _sc_kernels()
    xs = dispatch(hidden_states, pe1, po1)

    y = pl.pallas_call(
        _expert_body,
        grid_spec=pltpu.PrefetchScalarGridSpec(
            num_scalar_prefetch=1,
            grid=(NB, I // IBLK),
            in_specs=[
                pl.BlockSpec((B, H), lambda b, c, s: (_xidx(b, s), 0)),
                pl.BlockSpec((1, H, IBLK), lambda b, c, s: (s[b], 0, _cidx(b, c, s))),
                pl.BlockSpec((1, H, IBLK), lambda b, c, s: (s[b], 0, _cidx(b, c, s))),
                pl.BlockSpec((1, IBLK, H), lambda b, c, s: (s[b], _cidx(b, c, s), 0)),
            ],
            out_specs=pl.BlockSpec((B, H), lambda b, c, s: (b, 0)),
        ),
        out_shape=jax.ShapeDtypeStruct((NPAD, H), jnp.float32),
    )(smap, xs, w_gate, w_up, w_out)

    out = combine(y, pe1, po1, rp)
    return (out, hs)
